# baseline (speedup 1.0000x reference)
import functools

import jax
import jax.numpy as jnp
from jax import lax
from jax.experimental import pallas as pl
from jax.experimental.pallas import tpu as pltpu
from jax.experimental.pallas import tpu_sc as plsc

NF = 32
LANES = 16
NW = 32
TPW = 245
NTILES_FULL = 7812
TAIL_BASE = 999936
RING = 16
CAP = 16
PACKN = 128
TRASH = 16384


def _sc_phase1(batch):
    mesh = plsc.VectorSubcoreMesh(
        core_axis_name="c", subcore_axis_name="s", num_cores=2, num_subcores=16
    )
    num_cores = mesh.num_cores
    nvreg = batch // LANES
    staging = jax.ShapeDtypeStruct((batch + 1, 1, 128), jnp.float32)

    @functools.partial(
        pl.kernel,
        out_type=(staging, staging),
        mesh=mesh,
        compiler_params=pltpu.CompilerParams(
            needs_layout_passes=False, use_tc_tiling_on_sc=True
        ),
        scratch_types=[
            pltpu.VMEM((batch,), jnp.int32),
            pltpu.VMEM((TPW * CAP + 16,), jnp.int32),
            pltpu.VMEM((TPW * CAP + 16,), jnp.int32),
            pltpu.VMEM((TPW + 32,), jnp.int32),
            pltpu.VMEM((RING, NF, 128), jnp.float32),
            pltpu.VMEM((64, NF), jnp.float32),
            pltpu.VMEM((PACKN, 1, 128), jnp.float32),
            pltpu.VMEM((PACKN,), jnp.int32),
            pltpu.SemaphoreType.DMA((RING,)),
            pltpu.SemaphoreType.DMA,
        ],
    )
    def phase1(users_hbm, movies_hbm, ut_hbm, mt_hbm, utail_hbm, mtail_hbm,
               ue_out, me_out,
               idsv, bu, bp, cnt, ring, tailbuf, pack, posx, sem, ssem):
        wid = lax.axis_index("s") * num_cores + lax.axis_index("c")
        tbase = wid * TPW
        lanes = lax.iota(jnp.int32, LANES)
        ones = jnp.ones((LANES,), jnp.int32)
        zeros = jnp.zeros((LANES,), jnp.int32)
        nt = jnp.where(wid == NW - 1, NTILES_FULL - 31 * TPW + 1, TPW)
        nt_main = jnp.where(wid == NW - 1, NTILES_FULL - 31 * TPW, TPW)

        def reinit_posx():
            for q in range(PACKN // LANES):
                posx[pl.ds(q * LANES, LANES)] = jnp.full(
                    (LANES,), TRASH, jnp.int32)

        def flush(out_hbm):
            pltpu.async_copy(pack, out_hbm.at[posx], ssem).wait()
            reinit_posx()

        def do_table(ids_hbm, t_hbm, tail_hbm, out_hbm):
            pltpu.sync_copy(ids_hbm, idsv)
            for q in range((TPW + 32) // LANES):
                cnt[pl.ds(q * LANES, LANES)] = jnp.zeros((LANES,), jnp.int32)
            reinit_posx()

            def filt1(v):
                ids16 = idsv[pl.ds(v * LANES, LANES)]
                tl = (ids16 >> 7) - tbase
                inb = (tl >= 0) & (tl < nt)
                tc_ = jnp.clip(tl, 0, TPW - 1)
                rank, _ = plsc.scan_count(tc_, mask=inb)
                cb = plsc.load_gather(cnt, [tc_], mask=inb)
                slot = tc_ * CAP + jnp.minimum(cb + rank - 1, CAP - 1)
                plsc.store_scatter(bu, [slot], ids16, mask=inb)
                plsc.store_scatter(bp, [slot], v * LANES + lanes, mask=inb)
                plsc.addupdate_scatter(cnt, [tc_], ones, mask=inb)

            def filt(v, carry):
                filt1(v * 2)
                filt1(v * 2 + 1)
                return carry

            lax.fori_loop(0, nvreg // 2, filt, 0)

            for r in range(RING - 1):
                pltpu.async_copy(
                    t_hbm.at[:, pl.ds((tbase + r) * 128, 128)],
                    ring.at[r], sem.at[r])

            def bucket(t, tm, jc, from_tail):
                cvec = cnt[pl.ds(t, LANES)]
                n = cvec[0]
                need_flush = (jc + CAP > PACKN) & (n > 0)

                @pl.when(need_flush)
                def _():
                    flush(out_hbm)

                j = jnp.where(need_flush, 0, jc)

                @pl.when(n > 0)
                def _():
                    bu16 = bu[pl.ds(t * CAP, LANES)]
                    bp16 = bp[pl.ds(t * CAP, LANES)]
                    vmask = lanes < n
                    rows16 = j + lanes
                    plsc.store_scatter(posx, [rows16], bp16, mask=vmask)
                    if from_tail:
                        ul16 = bu16 - TAIL_BASE
                        for f in range(NF):
                            f16 = jnp.full((LANES,), f, jnp.int32)
                            vals = plsc.load_gather(
                                tailbuf, [ul16, f16], mask=vmask)
                            plsc.store_scatter(
                                pack, [rows16, zeros, f16], vals, mask=vmask)
                    else:
                        ul16 = bu16 & 127
                        tm16 = jnp.full((LANES,), tm, jnp.int32)
                        for f in range(NF):
                            f16 = jnp.full((LANES,), f, jnp.int32)
                            vals = plsc.load_gather(
                                ring, [tm16, f16, ul16], mask=vmask)
                            plsc.store_scatter(
                                pack, [rows16, zeros, f16], vals, mask=vmask)

                return j + n

            def tile_body(t, jc):
                tm = t % RING
                tn = (t + RING - 1) % RING

                @pl.when(t + RING - 1 < nt_main)
                def _():
                    pltpu.async_copy(
                        t_hbm.at[:, pl.ds((tbase + t + RING - 1) * 128, 128)],
                        ring.at[tn], sem.at[tn])

                pltpu.make_async_copy(
                    t_hbm.at[:, pl.ds(0, 128)], ring.at[tm], sem.at[tm]
                ).wait()
                return bucket(t, tm, jc, False)

            j = lax.fori_loop(0, nt_main, tile_body, 0)

            @pl.when(wid == NW - 1)
            def _():
                pltpu.sync_copy(tail_hbm, tailbuf)

            def tail_items(jj):
                return bucket(nt_main, 0, jj, True)

            j = lax.cond(wid == NW - 1, tail_items, lambda jj: jj, j)

            @pl.when(j > 0)
            def _():
                flush(out_hbm)

        do_table(users_hbm, ut_hbm, utail_hbm, ue_out)
        do_table(movies_hbm, mt_hbm, mtail_hbm, me_out)

    return phase1


def _tc_phase2(batch):
    blk = 1024
    grid = (batch // blk,)

    def body(u_ref, m_ref, o_ref):
        ue = u_ref[...][:, 0, :NF]
        me = m_ref[...][:, 0, :NF]
        dot = jnp.sum(ue * me, axis=1)
        un = jnp.sqrt(jnp.sum(ue * ue, axis=1))
        mn = jnp.sqrt(jnp.sum(me * me, axis=1))
        su = jnp.where(un > 1.0, 1.0 / (un + 1e-7), 1.0)
        sm = jnp.where(mn > 1.0, 1.0 / (mn + 1e-7), 1.0)
        denom = jnp.maximum(un * su * mn * sm, 1e-8)
        o_ref[...] = (dot * su * sm) / denom * 2.0 + 3.0

    return pl.pallas_call(
        body,
        grid=grid,
        in_specs=[
            pl.BlockSpec((blk, 1, 128), lambda i: (i, 0, 0)),
            pl.BlockSpec((blk, 1, 128), lambda i: (i, 0, 0)),
        ],
        out_specs=pl.BlockSpec((blk,), lambda i: (i,)),
        out_shape=jax.ShapeDtypeStruct((batch,), jnp.float32),
    )


def kernel(users, movies, user_table, movie_table):
    batch = users.shape[0]
    utail = user_table[TAIL_BASE:, :]
    mtail = movie_table[TAIL_BASE:, :]
    p1 = _sc_phase1(batch)
    ue_rows, me_rows = p1(users.astype(jnp.int32), movies.astype(jnp.int32),
                          user_table.T, movie_table.T, utail, mtail)
    return _tc_phase2(batch)(ue_rows, me_rows)


# speedup vs baseline: 1.0006x; 1.0006x over previous
import functools

import jax
import jax.numpy as jnp
from jax import lax
from jax.experimental import pallas as pl
from jax.experimental.pallas import tpu as pltpu
from jax.experimental.pallas import tpu_sc as plsc

NF = 32
LANES = 16
NW = 32
TPW = 245
NTILES_FULL = 7812
TAIL_BASE = 999936
RING = 16
CAP = 16
PACKN = 128
TRASH = 16384


def _sc_phase1(batch):
    mesh = plsc.VectorSubcoreMesh(
        core_axis_name="c", subcore_axis_name="s", num_cores=2, num_subcores=16
    )
    num_cores = mesh.num_cores
    nvreg = batch // LANES
    staging = jax.ShapeDtypeStruct((batch + 1, 1, 128), jnp.float32)

    @functools.partial(
        pl.kernel,
        out_type=(staging, staging),
        mesh=mesh,
        compiler_params=pltpu.CompilerParams(
            needs_layout_passes=False, use_tc_tiling_on_sc=True
        ),
        scratch_types=[
            pltpu.VMEM((batch,), jnp.int32),
            pltpu.VMEM((TPW * CAP + 16,), jnp.int32),
            pltpu.VMEM((TPW * CAP + 16,), jnp.int32),
            pltpu.VMEM((TPW + 32,), jnp.int32),
            pltpu.VMEM((RING, NF, 128), jnp.float32),
            pltpu.VMEM((64, NF), jnp.float32),
            pltpu.VMEM((PACKN, 1, 128), jnp.float32),
            pltpu.VMEM((PACKN,), jnp.int32),
            pltpu.SemaphoreType.DMA((RING,)),
            pltpu.SemaphoreType.DMA,
        ],
    )
    def phase1(users_hbm, movies_hbm, ut_hbm, mt_hbm, utail_hbm, mtail_hbm,
               ue_out, me_out,
               idsv, bu, bp, cnt, ring, tailbuf, pack, posx, sem, ssem):
        wid = lax.axis_index("s") * num_cores + lax.axis_index("c")
        tbase = wid * TPW
        lanes = lax.iota(jnp.int32, LANES)
        ones = jnp.ones((LANES,), jnp.int32)
        zeros = jnp.zeros((LANES,), jnp.int32)
        nt = jnp.where(wid == NW - 1, NTILES_FULL - 31 * TPW + 1, TPW)
        nt_main = jnp.where(wid == NW - 1, NTILES_FULL - 31 * TPW, TPW)

        def reinit_posx():
            for q in range(PACKN // LANES):
                posx[pl.ds(q * LANES, LANES)] = jnp.full(
                    (LANES,), TRASH, jnp.int32)

        def flush(out_hbm):
            pltpu.async_copy(pack, out_hbm.at[posx], ssem).wait()
            reinit_posx()

        def do_table(ids_hbm, t_hbm, tail_hbm, out_hbm):
            pltpu.sync_copy(ids_hbm, idsv)
            for q in range((TPW + 32) // LANES):
                cnt[pl.ds(q * LANES, LANES)] = jnp.zeros((LANES,), jnp.int32)
            reinit_posx()

            def filt1(v):
                ids16 = idsv[pl.ds(v * LANES, LANES)]
                tl = (ids16 >> 7) - tbase
                inb = (tl >= 0) & (tl < nt)
                tc_ = jnp.clip(tl, 0, TPW - 1)
                rank, _ = plsc.scan_count(tc_, mask=inb)
                cb = plsc.load_gather(cnt, [tc_], mask=inb)
                slot = tc_ * CAP + jnp.minimum(cb + rank - 1, CAP - 1)
                plsc.store_scatter(bu, [slot], ids16, mask=inb)
                plsc.store_scatter(bp, [slot], v * LANES + lanes, mask=inb)
                plsc.addupdate_scatter(cnt, [tc_], ones, mask=inb)

            def filt(v, carry):
                filt1(v * 2)
                filt1(v * 2 + 1)
                return carry

            lax.fori_loop(0, nvreg // 2, filt, 0)

            for r in range(RING - 1):
                pltpu.async_copy(
                    t_hbm.at[:, pl.ds((tbase + r) * 128, 128)],
                    ring.at[r], sem.at[r])

            def bucket(t, tm, jc, from_tail):
                cvec = cnt[pl.ds(t, LANES)]
                n = cvec[0]
                need_flush = (jc + CAP > PACKN) & (n > 0)

                @pl.when(need_flush)
                def _():
                    flush(out_hbm)

                j = jnp.where(need_flush, 0, jc)

                @pl.when(n > 0)
                def _():
                    bu16 = bu[pl.ds(t * CAP, LANES)]
                    bp16 = bp[pl.ds(t * CAP, LANES)]
                    vmask = lanes < n
                    rows16 = j + lanes
                    plsc.store_scatter(posx, [rows16], bp16, mask=vmask)
                    if from_tail:
                        ul16 = jnp.clip(bu16 - TAIL_BASE, 0, 63)
                        for f in range(NF):
                            f16 = jnp.full((LANES,), f, jnp.int32)
                            vals = plsc.load_gather(tailbuf, [ul16, f16])
                            plsc.store_scatter(
                                pack, [rows16, zeros, f16], vals, mask=vmask)
                    else:
                        ul16 = bu16 & 127
                        tm16 = jnp.full((LANES,), tm, jnp.int32)
                        for f in range(NF):
                            f16 = jnp.full((LANES,), f, jnp.int32)
                            vals = plsc.load_gather(ring, [tm16, f16, ul16])
                            plsc.store_scatter(
                                pack, [rows16, zeros, f16], vals, mask=vmask)

                return j + n

            def tile_body(t, jc):
                tm = t % RING
                tn = (t + RING - 1) % RING

                @pl.when(t + RING - 1 < nt_main)
                def _():
                    pltpu.async_copy(
                        t_hbm.at[:, pl.ds((tbase + t + RING - 1) * 128, 128)],
                        ring.at[tn], sem.at[tn])

                pltpu.make_async_copy(
                    t_hbm.at[:, pl.ds(0, 128)], ring.at[tm], sem.at[tm]
                ).wait()
                return bucket(t, tm, jc, False)

            j = lax.fori_loop(0, nt_main, tile_body, 0)

            @pl.when(wid == NW - 1)
            def _():
                pltpu.sync_copy(tail_hbm, tailbuf)

            def tail_items(jj):
                return bucket(nt_main, 0, jj, True)

            j = lax.cond(wid == NW - 1, tail_items, lambda jj: jj, j)

            @pl.when(j > 0)
            def _():
                flush(out_hbm)

        do_table(users_hbm, ut_hbm, utail_hbm, ue_out)
        do_table(movies_hbm, mt_hbm, mtail_hbm, me_out)

    return phase1


def _tc_phase2(batch):
    blk = 1024
    grid = (batch // blk,)

    def body(u_ref, m_ref, o_ref):
        ue = u_ref[...][:, 0, :NF]
        me = m_ref[...][:, 0, :NF]
        dot = jnp.sum(ue * me, axis=1)
        un = jnp.sqrt(jnp.sum(ue * ue, axis=1))
        mn = jnp.sqrt(jnp.sum(me * me, axis=1))
        su = jnp.where(un > 1.0, 1.0 / (un + 1e-7), 1.0)
        sm = jnp.where(mn > 1.0, 1.0 / (mn + 1e-7), 1.0)
        denom = jnp.maximum(un * su * mn * sm, 1e-8)
        o_ref[...] = (dot * su * sm) / denom * 2.0 + 3.0

    return pl.pallas_call(
        body,
        grid=grid,
        in_specs=[
            pl.BlockSpec((blk, 1, 128), lambda i: (i, 0, 0)),
            pl.BlockSpec((blk, 1, 128), lambda i: (i, 0, 0)),
        ],
        out_specs=pl.BlockSpec((blk,), lambda i: (i,)),
        out_shape=jax.ShapeDtypeStruct((batch,), jnp.float32),
    )


def kernel(users, movies, user_table, movie_table):
    batch = users.shape[0]
    utail = user_table[TAIL_BASE:, :]
    mtail = movie_table[TAIL_BASE:, :]
    p1 = _sc_phase1(batch)
    ue_rows, me_rows = p1(users.astype(jnp.int32), movies.astype(jnp.int32),
                          user_table.T, movie_table.T, utail, mtail)
    return _tc_phase2(batch)(ue_rows, me_rows)


# speedup vs baseline: 1.5253x; 1.5243x over previous
import functools

import jax
import jax.numpy as jnp
from jax import lax
from jax.experimental import pallas as pl
from jax.experimental.pallas import tpu as pltpu
from jax.experimental.pallas import tpu_sc as plsc

NF = 32
LANES = 16
NW = 32
TPW = 245
NTILES_FULL = 7812
TAIL_BASE = 999936
RING = 16
CAP = 16
PACKN = 128
TRASH = 16384


def _sc_phase1(batch):
    mesh = plsc.VectorSubcoreMesh(
        core_axis_name="c", subcore_axis_name="s", num_cores=2, num_subcores=16
    )
    num_cores = mesh.num_cores
    nvreg = batch // LANES
    staging = jax.ShapeDtypeStruct((batch + 1, 1, 128), jnp.float32)

    @functools.partial(
        pl.kernel,
        out_type=(staging, staging),
        mesh=mesh,
        compiler_params=pltpu.CompilerParams(
            needs_layout_passes=False, use_tc_tiling_on_sc=True
        ),
        scratch_types=[
            pltpu.VMEM((batch,), jnp.int32),
            pltpu.VMEM((TPW * CAP + 16,), jnp.int32),
            pltpu.VMEM((TPW * CAP + 16,), jnp.int32),
            pltpu.VMEM((TPW + 32,), jnp.int32),
            pltpu.VMEM((RING, NF, 128), jnp.float32),
            pltpu.VMEM((64, NF), jnp.float32),
            pltpu.VMEM((PACKN, 1, 128), jnp.float32),
            pltpu.VMEM((PACKN,), jnp.int32),
            pltpu.SemaphoreType.DMA((RING,)),
            pltpu.SemaphoreType.DMA,
        ],
    )
    def phase1(users_hbm, movies_hbm, ut_hbm, mt_hbm, utail_hbm, mtail_hbm,
               ue_out, me_out,
               idsv, bu, bp, cnt, ring, tailbuf, pack, posx, sem, ssem):
        wid = lax.axis_index("s") * num_cores + lax.axis_index("c")
        tbase = wid * TPW
        lanes = lax.iota(jnp.int32, LANES)
        ones = jnp.ones((LANES,), jnp.int32)
        lane0 = lanes == 0
        nt = jnp.where(wid == NW - 1, NTILES_FULL - 31 * TPW + 1, TPW)
        nt_main = jnp.where(wid == NW - 1, NTILES_FULL - 31 * TPW, TPW)

        def reinit_posx():
            for q in range(PACKN // LANES):
                posx[pl.ds(q * LANES, LANES)] = jnp.full(
                    (LANES,), TRASH, jnp.int32)

        def flush(out_hbm):
            pltpu.async_copy(pack, out_hbm.at[posx], ssem).wait()
            reinit_posx()

        def do_table(ids_hbm, t_hbm, tail_hbm, out_hbm):
            pltpu.sync_copy(ids_hbm, idsv)
            for q in range((TPW + 32) // LANES):
                cnt[pl.ds(q * LANES, LANES)] = jnp.zeros((LANES,), jnp.int32)
            reinit_posx()

            def filt(v, carry):
                ids16 = idsv[pl.ds(v * LANES, LANES)]
                tl = (ids16 >> 7) - tbase
                inb = (tl >= 0) & (tl < nt)
                tc_ = jnp.clip(tl, 0, TPW - 1)
                rank, _ = plsc.scan_count(tc_, mask=inb)
                cb = plsc.load_gather(cnt, [tc_], mask=inb)
                slot = tc_ * CAP + jnp.minimum(cb + rank - 1, CAP - 1)
                plsc.store_scatter(bu, [slot], ids16, mask=inb)
                plsc.store_scatter(bp, [slot], v * LANES + lanes, mask=inb)
                plsc.addupdate_scatter(cnt, [tc_], ones, mask=inb)
                return carry

            lax.fori_loop(0, nvreg, filt, 0)

            for r in range(RING - 1):
                pltpu.async_copy(
                    t_hbm.at[:, pl.ds((tbase + r) * 128, 128)],
                    ring.at[r], sem.at[r])

            def item(i, jc, t, tm, from_tail):
                j = jc
                u = plsc.load_gather(bu, [jnp.full((LANES,), t * CAP + i,
                                                   jnp.int32)])[0]
                pos = plsc.load_gather(bp, [jnp.full((LANES,), t * CAP + i,
                                                     jnp.int32)])[0]
                ul = u & 127
                if from_tail:
                    uf0 = plsc.load_gather(
                        tailbuf, [jnp.full((LANES,), u - TAIL_BASE, jnp.int32),
                                  lanes])
                    uf1 = plsc.load_gather(
                        tailbuf, [jnp.full((LANES,), u - TAIL_BASE, jnp.int32),
                                  lanes + LANES])
                else:
                    uf0 = plsc.load_gather(
                        ring, [jnp.full((LANES,), tm, jnp.int32), lanes,
                               jnp.full((LANES,), ul, jnp.int32)])
                    uf1 = plsc.load_gather(
                        ring, [jnp.full((LANES,), tm, jnp.int32),
                               lanes + LANES,
                               jnp.full((LANES,), ul, jnp.int32)])
                pack[j, 0, pl.ds(0, LANES)] = uf0
                pack[j, 0, pl.ds(LANES, LANES)] = uf1
                plsc.store_scatter(posx, [jnp.full((LANES,), j, jnp.int32)],
                                   jnp.full((LANES,), pos, jnp.int32),
                                   mask=lane0)
                j = j + 1

                @pl.when(j == PACKN)
                def _():
                    flush(out_hbm)

                return jnp.where(j == PACKN, 0, j)

            def tile_body(t, jc):
                tm = t % RING
                tn = (t + RING - 1) % RING

                @pl.when(t + RING - 1 < nt_main)
                def _():
                    pltpu.async_copy(
                        t_hbm.at[:, pl.ds((tbase + t + RING - 1) * 128, 128)],
                        ring.at[tn], sem.at[tn])

                pltpu.make_async_copy(
                    t_hbm.at[:, pl.ds(0, 128)], ring.at[tm], sem.at[tm]
                ).wait()
                cvec = cnt[pl.ds(t, LANES)]
                n = cvec[0]
                jc = lax.fori_loop(
                    0, n, lambda i, j: item(i, j, t, tm, False), jc)
                return jc

            j = lax.fori_loop(0, nt_main, tile_body, 0)

            @pl.when(wid == NW - 1)
            def _():
                pltpu.sync_copy(tail_hbm, tailbuf)

            def tail_items(jj):
                t = nt_main
                cvec = cnt[pl.ds(t, LANES)]
                n = cvec[0]
                return lax.fori_loop(
                    0, n, lambda i, j2: item(i, j2, t, 0, True), jj)

            j = lax.cond(wid == NW - 1, tail_items, lambda jj: jj, j)

            @pl.when(j > 0)
            def _():
                flush(out_hbm)

        do_table(users_hbm, ut_hbm, utail_hbm, ue_out)
        do_table(movies_hbm, mt_hbm, mtail_hbm, me_out)

    return phase1


def _tc_phase2(batch):
    blk = 1024
    grid = (batch // blk,)

    def body(u_ref, m_ref, o_ref):
        ue = u_ref[...][:, 0, :NF]
        me = m_ref[...][:, 0, :NF]
        dot = jnp.sum(ue * me, axis=1)
        un = jnp.sqrt(jnp.sum(ue * ue, axis=1))
        mn = jnp.sqrt(jnp.sum(me * me, axis=1))
        su = jnp.where(un > 1.0, 1.0 / (un + 1e-7), 1.0)
        sm = jnp.where(mn > 1.0, 1.0 / (mn + 1e-7), 1.0)
        denom = jnp.maximum(un * su * mn * sm, 1e-8)
        o_ref[...] = (dot * su * sm) / denom * 2.0 + 3.0

    return pl.pallas_call(
        body,
        grid=grid,
        in_specs=[
            pl.BlockSpec((blk, 1, 128), lambda i: (i, 0, 0)),
            pl.BlockSpec((blk, 1, 128), lambda i: (i, 0, 0)),
        ],
        out_specs=pl.BlockSpec((blk,), lambda i: (i,)),
        out_shape=jax.ShapeDtypeStruct((batch,), jnp.float32),
    )


def kernel(users, movies, user_table, movie_table):
    batch = users.shape[0]
    utail = user_table[TAIL_BASE:, :]
    mtail = movie_table[TAIL_BASE:, :]
    p1 = _sc_phase1(batch)
    ue_rows, me_rows = p1(users.astype(jnp.int32), movies.astype(jnp.int32),
                          user_table.T, movie_table.T, utail, mtail)
    return _tc_phase2(batch)(ue_rows, me_rows)


# speedup vs baseline: 1.5280x; 1.0018x over previous
import functools

import jax
import jax.numpy as jnp
from jax import lax
from jax.experimental import pallas as pl
from jax.experimental.pallas import tpu as pltpu
from jax.experimental.pallas import tpu_sc as plsc

NF = 32
LANES = 16
NW = 32
TPW = 245
NTILES_FULL = 7812
TAIL_BASE = 999936
RING = 16
CAP = 16
PACKN = 128
TRASH = 16384


def _sc_phase1(batch):
    mesh = plsc.VectorSubcoreMesh(
        core_axis_name="c", subcore_axis_name="s", num_cores=2, num_subcores=16
    )
    num_cores = mesh.num_cores
    nvreg = batch // LANES
    staging = jax.ShapeDtypeStruct((batch + 1, 1, 128), jnp.float32)

    @functools.partial(
        pl.kernel,
        out_type=(staging, staging),
        mesh=mesh,
        compiler_params=pltpu.CompilerParams(
            needs_layout_passes=False, use_tc_tiling_on_sc=True
        ),
        scratch_types=[
            pltpu.VMEM((batch,), jnp.int32),
            pltpu.VMEM((TPW * CAP + 16,), jnp.int32),
            pltpu.VMEM((TPW * CAP + 16,), jnp.int32),
            pltpu.VMEM((TPW + 32,), jnp.int32),
            pltpu.VMEM((RING, NF, 128), jnp.float32),
            pltpu.VMEM((64, NF), jnp.float32),
            pltpu.VMEM((PACKN, 1, 128), jnp.float32),
            pltpu.VMEM((PACKN,), jnp.int32),
            pltpu.SemaphoreType.DMA((RING,)),
            pltpu.SemaphoreType.DMA,
        ],
    )
    def phase1(users_hbm, movies_hbm, ut_hbm, mt_hbm, utail_hbm, mtail_hbm,
               ue_out, me_out,
               idsv, bu, bp, cnt, ring, tailbuf, pack, posx, sem, ssem):
        wid = lax.axis_index("s") * num_cores + lax.axis_index("c")
        tbase = wid * TPW
        lanes = lax.iota(jnp.int32, LANES)
        ones = jnp.ones((LANES,), jnp.int32)
        lane0 = lanes == 0
        nt = jnp.where(wid == NW - 1, NTILES_FULL - 31 * TPW + 1, TPW)
        nt_main = jnp.where(wid == NW - 1, NTILES_FULL - 31 * TPW, TPW)

        def reinit_posx():
            for q in range(PACKN // LANES):
                posx[pl.ds(q * LANES, LANES)] = jnp.full(
                    (LANES,), TRASH, jnp.int32)

        def flush(out_hbm):
            pltpu.async_copy(pack, out_hbm.at[posx], ssem).wait()
            reinit_posx()

        def do_table(ids_hbm, t_hbm, tail_hbm, out_hbm):
            pltpu.sync_copy(ids_hbm, idsv)
            for q in range((TPW + 32) // LANES):
                cnt[pl.ds(q * LANES, LANES)] = jnp.zeros((LANES,), jnp.int32)
            reinit_posx()

            def filt1(v):
                ids16 = idsv[pl.ds(v * LANES, LANES)]
                tl = (ids16 >> 7) - tbase
                inb = (tl >= 0) & (tl < nt)
                tc_ = jnp.clip(tl, 0, TPW - 1)
                rank, _ = plsc.scan_count(tc_, mask=inb)
                cb = plsc.load_gather(cnt, [tc_], mask=inb)
                slot = tc_ * CAP + jnp.minimum(cb + rank - 1, CAP - 1)
                plsc.store_scatter(bu, [slot], ids16, mask=inb)
                plsc.store_scatter(bp, [slot], v * LANES + lanes, mask=inb)
                plsc.addupdate_scatter(cnt, [tc_], ones, mask=inb)

            def filt(v, carry):
                filt1(v * 2)
                filt1(v * 2 + 1)
                return carry

            lax.fori_loop(0, nvreg // 2, filt, 0)


            for r in range(RING - 1):
                pltpu.async_copy(
                    t_hbm.at[:, pl.ds((tbase + r) * 128, 128)],
                    ring.at[r], sem.at[r])

            def item(i, jc, t, tm, from_tail):
                j = jc
                u = plsc.load_gather(bu, [jnp.full((LANES,), t * CAP + i,
                                                   jnp.int32)])[0]
                pos = plsc.load_gather(bp, [jnp.full((LANES,), t * CAP + i,
                                                     jnp.int32)])[0]
                ul = u & 127
                if from_tail:
                    uf0 = plsc.load_gather(
                        tailbuf, [jnp.full((LANES,), u - TAIL_BASE, jnp.int32),
                                  lanes])
                    uf1 = plsc.load_gather(
                        tailbuf, [jnp.full((LANES,), u - TAIL_BASE, jnp.int32),
                                  lanes + LANES])
                else:
                    uf0 = plsc.load_gather(
                        ring, [jnp.full((LANES,), tm, jnp.int32), lanes,
                               jnp.full((LANES,), ul, jnp.int32)])
                    uf1 = plsc.load_gather(
                        ring, [jnp.full((LANES,), tm, jnp.int32),
                               lanes + LANES,
                               jnp.full((LANES,), ul, jnp.int32)])
                pack[j, 0, pl.ds(0, LANES)] = uf0
                pack[j, 0, pl.ds(LANES, LANES)] = uf1
                plsc.store_scatter(posx, [jnp.full((LANES,), j, jnp.int32)],
                                   jnp.full((LANES,), pos, jnp.int32),
                                   mask=lane0)
                j = j + 1

                @pl.when(j == PACKN)
                def _():
                    flush(out_hbm)

                return jnp.where(j == PACKN, 0, j)

            def tile_body(t, jc):
                tm = t % RING
                tn = (t + RING - 1) % RING

                @pl.when(t + RING - 1 < nt_main)
                def _():
                    pltpu.async_copy(
                        t_hbm.at[:, pl.ds((tbase + t + RING - 1) * 128, 128)],
                        ring.at[tn], sem.at[tn])

                pltpu.make_async_copy(
                    t_hbm.at[:, pl.ds(0, 128)], ring.at[tm], sem.at[tm]
                ).wait()
                cvec = cnt[pl.ds(t, LANES)]
                n = cvec[0]
                jc = lax.fori_loop(
                    0, n, lambda i, j: item(i, j, t, tm, False), jc)
                return jc

            j = lax.fori_loop(0, nt_main, tile_body, 0)

            @pl.when(wid == NW - 1)
            def _():
                pltpu.sync_copy(tail_hbm, tailbuf)

            def tail_items(jj):
                t = nt_main
                cvec = cnt[pl.ds(t, LANES)]
                n = cvec[0]
                return lax.fori_loop(
                    0, n, lambda i, j2: item(i, j2, t, 0, True), jj)

            j = lax.cond(wid == NW - 1, tail_items, lambda jj: jj, j)

            @pl.when(j > 0)
            def _():
                flush(out_hbm)

        do_table(users_hbm, ut_hbm, utail_hbm, ue_out)
        do_table(movies_hbm, mt_hbm, mtail_hbm, me_out)

    return phase1


def _tc_phase2(batch):
    blk = 1024
    grid = (batch // blk,)

    def body(u_ref, m_ref, o_ref):
        ue = u_ref[...][:, 0, :NF]
        me = m_ref[...][:, 0, :NF]
        dot = jnp.sum(ue * me, axis=1)
        un = jnp.sqrt(jnp.sum(ue * ue, axis=1))
        mn = jnp.sqrt(jnp.sum(me * me, axis=1))
        su = jnp.where(un > 1.0, 1.0 / (un + 1e-7), 1.0)
        sm = jnp.where(mn > 1.0, 1.0 / (mn + 1e-7), 1.0)
        denom = jnp.maximum(un * su * mn * sm, 1e-8)
        o_ref[...] = (dot * su * sm) / denom * 2.0 + 3.0

    return pl.pallas_call(
        body,
        grid=grid,
        in_specs=[
            pl.BlockSpec((blk, 1, 128), lambda i: (i, 0, 0)),
            pl.BlockSpec((blk, 1, 128), lambda i: (i, 0, 0)),
        ],
        out_specs=pl.BlockSpec((blk,), lambda i: (i,)),
        out_shape=jax.ShapeDtypeStruct((batch,), jnp.float32),
    )


def kernel(users, movies, user_table, movie_table):
    batch = users.shape[0]
    utail = user_table[TAIL_BASE:, :]
    mtail = movie_table[TAIL_BASE:, :]
    p1 = _sc_phase1(batch)
    ue_rows, me_rows = p1(users.astype(jnp.int32), movies.astype(jnp.int32),
                          user_table.T, movie_table.T, utail, mtail)
    return _tc_phase2(batch)(ue_rows, me_rows)


# speedup vs baseline: 1.5745x; 1.0304x over previous
import functools

import jax
import jax.numpy as jnp
from jax import lax
from jax.experimental import pallas as pl
from jax.experimental.pallas import tpu as pltpu
from jax.experimental.pallas import tpu_sc as plsc

NF = 32
LANES = 16
NW = 32
TPW = 245
NTILES_FULL = 7812
TAIL_BASE = 999936
RING = 8
CAP = 16
PACKN = 128
TRASH = 16384


def _sc_phase1(batch):
    mesh = plsc.VectorSubcoreMesh(
        core_axis_name="c", subcore_axis_name="s", num_cores=2, num_subcores=16
    )
    num_cores = mesh.num_cores
    nvreg = batch // LANES
    staging = jax.ShapeDtypeStruct((batch + 1, 1, 128), jnp.float32)

    @functools.partial(
        pl.kernel,
        out_type=(staging, staging),
        mesh=mesh,
        compiler_params=pltpu.CompilerParams(
            needs_layout_passes=False, use_tc_tiling_on_sc=True
        ),
        scratch_types=[
            pltpu.VMEM((batch,), jnp.int32),
            pltpu.VMEM((TPW * CAP + 16,), jnp.int32),
            pltpu.VMEM((TPW * CAP + 16,), jnp.int32),
            pltpu.VMEM((TPW + 32,), jnp.int32),
            pltpu.VMEM((RING, NF, 256), jnp.float32),
            pltpu.VMEM((64, NF), jnp.float32),
            pltpu.VMEM((PACKN, 1, 128), jnp.float32),
            pltpu.VMEM((PACKN,), jnp.int32),
            pltpu.SemaphoreType.DMA((RING,)),
            pltpu.SemaphoreType.DMA,
        ],
    )
    def phase1(users_hbm, movies_hbm, ut_hbm, mt_hbm, utail_hbm, mtail_hbm,
               ue_out, me_out,
               idsv, bu, bp, cnt, ring, tailbuf, pack, posx, sem, ssem):
        wid = lax.axis_index("s") * num_cores + lax.axis_index("c")
        tbase = wid * TPW
        lanes = lax.iota(jnp.int32, LANES)
        ones = jnp.ones((LANES,), jnp.int32)
        lane0 = lanes == 0
        nt = jnp.where(wid == NW - 1, NTILES_FULL - 31 * TPW + 1, TPW)
        nt_main = jnp.where(wid == NW - 1, NTILES_FULL - 31 * TPW, TPW)

        def reinit_posx():
            for q in range(PACKN // LANES):
                posx[pl.ds(q * LANES, LANES)] = jnp.full(
                    (LANES,), TRASH, jnp.int32)

        def flush(out_hbm):
            pltpu.async_copy(pack, out_hbm.at[posx], ssem).wait()
            reinit_posx()

        def do_table(ids_hbm, t_hbm, tail_hbm, out_hbm):
            pltpu.sync_copy(ids_hbm, idsv)
            for q in range((TPW + 32) // LANES):
                cnt[pl.ds(q * LANES, LANES)] = jnp.zeros((LANES,), jnp.int32)
            reinit_posx()

            def filt1(v):
                ids16 = idsv[pl.ds(v * LANES, LANES)]
                tl = (ids16 >> 7) - tbase
                inb = (tl >= 0) & (tl < nt)
                tc_ = jnp.clip(tl, 0, TPW - 1)
                rank, _ = plsc.scan_count(tc_, mask=inb)
                cb = plsc.load_gather(cnt, [tc_], mask=inb)
                slot = tc_ * CAP + jnp.minimum(cb + rank - 1, CAP - 1)
                plsc.store_scatter(bu, [slot], ids16, mask=inb)
                plsc.store_scatter(bp, [slot], v * LANES + lanes, mask=inb)
                plsc.addupdate_scatter(cnt, [tc_], ones, mask=inb)

            def filt(v, carry):
                filt1(v * 2)
                filt1(v * 2 + 1)
                return carry

            lax.fori_loop(0, nvreg // 2, filt, 0)


            nq = nt_main // 2
            for r in range(RING - 1):
                pltpu.async_copy(
                    t_hbm.at[:, pl.ds((tbase + 2 * r) * 128, 256)],
                    ring.at[r], sem.at[r])

            def item(i, jc, t, tm, from_tail, off=0):
                j = jc
                u = plsc.load_gather(bu, [jnp.full((LANES,), t * CAP + i,
                                                   jnp.int32)])[0]
                pos = plsc.load_gather(bp, [jnp.full((LANES,), t * CAP + i,
                                                     jnp.int32)])[0]
                ul = u & 127
                if from_tail:
                    uf0 = plsc.load_gather(
                        tailbuf, [jnp.full((LANES,), u - TAIL_BASE, jnp.int32),
                                  lanes])
                    uf1 = plsc.load_gather(
                        tailbuf, [jnp.full((LANES,), u - TAIL_BASE, jnp.int32),
                                  lanes + LANES])
                else:
                    uf0 = plsc.load_gather(
                        ring, [jnp.full((LANES,), tm, jnp.int32), lanes,
                               jnp.full((LANES,), ul + off, jnp.int32)])
                    uf1 = plsc.load_gather(
                        ring, [jnp.full((LANES,), tm, jnp.int32),
                               lanes + LANES,
                               jnp.full((LANES,), ul + off, jnp.int32)])
                pack[j, 0, pl.ds(0, LANES)] = uf0
                pack[j, 0, pl.ds(LANES, LANES)] = uf1
                plsc.store_scatter(posx, [jnp.full((LANES,), j, jnp.int32)],
                                   jnp.full((LANES,), pos, jnp.int32),
                                   mask=lane0)
                j = j + 1

                @pl.when(j == PACKN)
                def _():
                    flush(out_hbm)

                return jnp.where(j == PACKN, 0, j)

            def proc_tile(t, tm, jc, off):
                cvec = cnt[pl.ds(t, LANES)]
                n = cvec[0]
                return lax.fori_loop(
                    0, n, lambda i, j: item(i, j, t, tm, False, off), jc)

            def tile_body(q, jc):
                tm = q % RING
                tn = (q + RING - 1) % RING

                @pl.when(q + RING - 1 < nq)
                def _():
                    pltpu.async_copy(
                        t_hbm.at[:, pl.ds((tbase + 2 * (q + RING - 1)) * 128,
                                          256)],
                        ring.at[tn], sem.at[tn])

                pltpu.make_async_copy(
                    t_hbm.at[:, pl.ds(0, 256)], ring.at[tm], sem.at[tm]
                ).wait()
                jc = proc_tile(2 * q, tm, jc, 0)
                jc = proc_tile(2 * q + 1, tm, jc, 128)
                return jc

            j = lax.fori_loop(0, nq, tile_body, 0)

            t_last = nt_main - 1
            pltpu.sync_copy(
                t_hbm.at[:, pl.ds((tbase + t_last) * 128, 128)],
                ring.at[0, :, pl.ds(0, 128)])
            j = proc_tile(t_last, 0, j, 0)

            @pl.when(wid == NW - 1)
            def _():
                pltpu.sync_copy(tail_hbm, tailbuf)

            def tail_items(jj):
                t = nt_main
                cvec = cnt[pl.ds(t, LANES)]
                n = cvec[0]
                return lax.fori_loop(
                    0, n, lambda i, j2: item(i, j2, t, 0, True), jj)

            j = lax.cond(wid == NW - 1, tail_items, lambda jj: jj, j)

            @pl.when(j > 0)
            def _():
                flush(out_hbm)

        do_table(users_hbm, ut_hbm, utail_hbm, ue_out)
        do_table(movies_hbm, mt_hbm, mtail_hbm, me_out)

    return phase1


def _tc_phase2(batch):
    blk = 1024
    grid = (batch // blk,)

    def body(u_ref, m_ref, o_ref):
        ue = u_ref[...][:, 0, :NF]
        me = m_ref[...][:, 0, :NF]
        dot = jnp.sum(ue * me, axis=1)
        un = jnp.sqrt(jnp.sum(ue * ue, axis=1))
        mn = jnp.sqrt(jnp.sum(me * me, axis=1))
        su = jnp.where(un > 1.0, 1.0 / (un + 1e-7), 1.0)
        sm = jnp.where(mn > 1.0, 1.0 / (mn + 1e-7), 1.0)
        denom = jnp.maximum(un * su * mn * sm, 1e-8)
        o_ref[...] = (dot * su * sm) / denom * 2.0 + 3.0

    return pl.pallas_call(
        body,
        grid=grid,
        in_specs=[
            pl.BlockSpec((blk, 1, 128), lambda i: (i, 0, 0)),
            pl.BlockSpec((blk, 1, 128), lambda i: (i, 0, 0)),
        ],
        out_specs=pl.BlockSpec((blk,), lambda i: (i,)),
        out_shape=jax.ShapeDtypeStruct((batch,), jnp.float32),
    )


def kernel(users, movies, user_table, movie_table):
    batch = users.shape[0]
    utail = user_table[TAIL_BASE:, :]
    mtail = movie_table[TAIL_BASE:, :]
    p1 = _sc_phase1(batch)
    ue_rows, me_rows = p1(users.astype(jnp.int32), movies.astype(jnp.int32),
                          user_table.T, movie_table.T, utail, mtail)
    return _tc_phase2(batch)(ue_rows, me_rows)
